# initial kernel scaffold (unmeasured)
import jax
import jax.numpy as jnp
from jax import lax
from jax.experimental import pallas as pl
from jax.experimental.pallas import tpu as pltpu

N_DEV = 4
B, S, D = 2, 512, 2048
H, DH, DR = 16, 128, 32
DC = 512
DCL = DC // N_DEV
BS = B * S
SCALE = (DH + DR) ** -0.5

_DevId = getattr(pl, "DeviceIdType", None) or pltpu.DeviceIdType
_sem_signal = getattr(pl, "semaphore_signal", None) or pltpu.semaphore_signal
_sem_wait = getattr(pl, "semaphore_wait", None) or pltpu.semaphore_wait
_CompilerParams = getattr(pltpu, "CompilerParams", None) or pltpu.TPUCompilerParams


def _dot(a, b, dims):
    return lax.dot_general(a, b, (dims, ((), ())),
                           preferred_element_type=jnp.float32)


def kernel(x, Wdkv, Wuk, Wuv, Wq, Wqr, Wkr, Wo):
    f = jnp.bfloat16
    xb = x.reshape(BS, D).astype(f)

    def body(x_ref, wdkv_ref, wuk_ref, wuv_ref, wq_ref, wqr_ref, wkr_ref,
             wo_ref, out_ref,
             ct_ref, ku_ref, vu_ref, q_ref, qr_ref, kr_ref,
             k_ref, v_ref, o_ref, send_sems, recv_sems):
        my = lax.axis_index("i")
        left = (my - 1) % N_DEV
        right = (my + 1) % N_DEV

        barrier = pltpu.get_barrier_semaphore()
        for nbr in (left, right):
            _sem_signal(barrier, inc=1, device_id=(nbr,),
                        device_id_type=_DevId.MESH)
        _sem_wait(barrier, 2)

        xv = x_ref[...]
        ct = _dot(wdkv_ref[...], xv, ((0,), (1,))).astype(f)
        ct_ref[pl.ds(my * DCL, DCL)] = ct
        ku_ref[pl.ds(my * DCL, DCL)] = wuk_ref[...]
        vu_ref[pl.ds(my * DCL, DCL)] = wuv_ref[...]

        for h in range(N_DEV - 1):
            origin = (my - h) % N_DEV
            rdmas = []
            for t, buf in enumerate((ct_ref, ku_ref, vu_ref)):
                r = pltpu.make_async_remote_copy(
                    src_ref=buf.at[pl.ds(origin * DCL, DCL)],
                    dst_ref=buf.at[pl.ds(origin * DCL, DCL)],
                    send_sem=send_sems.at[t, h],
                    recv_sem=recv_sems.at[t, h],
                    device_id=(right,),
                    device_id_type=_DevId.MESH,
                )
                r.start()
                rdmas.append(r)
            if h == 0:
                q_ref[...] = _dot(xv, wq_ref[...], ((1,), (0,))).astype(f)
                qr_ref[...] = _dot(xv, wqr_ref[...], ((1,), (0,))).astype(f)
                kr_ref[...] = _dot(xv, wkr_ref[...], ((1,), (0,))).astype(f)
            for r in rdmas:
                r.wait()

        k_ref[...] = _dot(ct_ref[...], ku_ref[...], ((0,), (0,))).astype(f)
        v_ref[...] = _dot(ct_ref[...], vu_ref[...], ((0,), (0,))).astype(f)

        for b in range(B):
            rows = slice(b * S, (b + 1) * S)
            krb = kr_ref[rows, :]
            for hh in range(H):
                cols = slice(hh * DH, (hh + 1) * DH)
                rcols = slice(hh * DR, (hh + 1) * DR)
                s = _dot(q_ref[rows, cols], k_ref[rows, cols], ((1,), (1,)))
                s = s + _dot(qr_ref[rows, rcols], krb, ((1,), (1,)))
                s = s * SCALE
                s = s - jnp.max(s, axis=-1, keepdims=True)
                p = jnp.exp(s)
                p = p / jnp.sum(p, axis=-1, keepdims=True)
                o = _dot(p.astype(f), v_ref[rows, cols], ((1,), (0,)))
                o_ref[rows, cols] = o.astype(f)

        out = _dot(o_ref[...], wo_ref[...], ((1,), (0,)))
        out_ref[...] = out.reshape(B, S, D)

    return pl.pallas_call(
        body,
        out_shape=jax.ShapeDtypeStruct((B, S, D), jnp.float32),
        in_specs=[pl.BlockSpec(memory_space=pltpu.VMEM)] * 8,
        out_specs=pl.BlockSpec(memory_space=pltpu.VMEM),
        scratch_shapes=[
            pltpu.VMEM((DC, BS), f),
            pltpu.VMEM((DC, D), f),
            pltpu.VMEM((DC, D), f),
            pltpu.VMEM((BS, D), f),
            pltpu.VMEM((BS, H * DR), f),
            pltpu.VMEM((BS, DR), f),
            pltpu.VMEM((BS, D), f),
            pltpu.VMEM((BS, D), f),
            pltpu.VMEM((BS, D), f),
            pltpu.SemaphoreType.DMA((3, N_DEV - 1)),
            pltpu.SemaphoreType.DMA((3, N_DEV - 1)),
        ],
        compiler_params=_CompilerParams(collective_id=0),
    )(xb, *(w.astype(f) for w in (Wdkv, Wuk, Wuv, Wq, Wqr, Wkr, Wo)))


# baseline (device time: 157043 ns/iter reference)
import jax
import jax.numpy as jnp
from jax import lax
from jax.experimental import pallas as pl
from jax.experimental.pallas import tpu as pltpu

N_DEV = 4
B, S, D = 2, 512, 2048
H, DH, DR = 16, 128, 32
DC = 512
DCL = DC // N_DEV
BS = B * S
SCALE = (DH + DR) ** -0.5

_DevId = getattr(pl, "DeviceIdType", None) or pltpu.DeviceIdType
_sem_signal = getattr(pl, "semaphore_signal", None) or pltpu.semaphore_signal
_sem_wait = getattr(pl, "semaphore_wait", None) or pltpu.semaphore_wait
_CompilerParams = getattr(pltpu, "CompilerParams", None) or pltpu.TPUCompilerParams


def _dot(a, b, dims):
    return lax.dot_general(a, b, (dims, ((), ())),
                           preferred_element_type=jnp.float32)


def kernel(x, Wdkv, Wuk, Wuv, Wq, Wqr, Wkr, Wo):
    f = jnp.bfloat16
    xb = x.reshape(BS, D).astype(f)

    def body(x_ref, wdkv_ref, wuk_ref, wuv_ref, wq_ref, wqr_ref, wkr_ref,
             wo_ref, out_ref,
             ct_ref, ku_ref, vu_ref, q_ref, qr_ref, kr_ref,
             k_ref, v_ref, send_sems, recv_sems):
        my = lax.axis_index("i")
        left = (my - 1) % N_DEV
        right = (my + 1) % N_DEV

        barrier = pltpu.get_barrier_semaphore()
        for nbr in (left, right):
            _sem_signal(barrier, inc=1, device_id=(nbr,),
                        device_id_type=_DevId.MESH)
        _sem_wait(barrier, 2)

        xv = x_ref[...]
        ct = _dot(wdkv_ref[...], xv, ((0,), (1,))).astype(f)
        ct_ref[pl.ds(my * DCL, DCL)] = ct
        ku_ref[pl.ds(my * DCL, DCL)] = wuk_ref[...]
        vu_ref[pl.ds(my * DCL, DCL)] = wuv_ref[...]

        for h in range(N_DEV - 1):
            origin = (my - h) % N_DEV
            rdmas = []
            for t, buf in enumerate((ct_ref, ku_ref, vu_ref)):
                r = pltpu.make_async_remote_copy(
                    src_ref=buf.at[pl.ds(origin * DCL, DCL)],
                    dst_ref=buf.at[pl.ds(origin * DCL, DCL)],
                    send_sem=send_sems.at[t, h],
                    recv_sem=recv_sems.at[t, h],
                    device_id=(right,),
                    device_id_type=_DevId.MESH,
                )
                r.start()
                rdmas.append(r)
            if h == 0:
                q_ref[...] = _dot(xv, wq_ref[...], ((1,), (0,))).astype(f)
                qr_ref[...] = _dot(xv, wqr_ref[...], ((1,), (0,))).astype(f)
                kr_ref[...] = _dot(xv, wkr_ref[...], ((1,), (0,))).astype(f)
            for r in rdmas:
                r.wait()

        k_ref[...] = _dot(ct_ref[...], ku_ref[...], ((0,), (0,))).astype(f)
        v_ref[...] = _dot(ct_ref[...], vu_ref[...], ((0,), (0,))).astype(f)

        for b in range(B):
            rows = slice(b * S, (b + 1) * S)
            krb = kr_ref[rows, :]
            acc = jnp.zeros((S, D), jnp.float32)
            for hh in range(H):
                cols = slice(hh * DH, (hh + 1) * DH)
                rcols = slice(hh * DR, (hh + 1) * DR)
                s = _dot(q_ref[rows, cols], k_ref[rows, cols], ((1,), (1,)))
                s = s + _dot(qr_ref[rows, rcols], krb, ((1,), (1,)))
                s = s * SCALE
                s = s - jnp.max(s, axis=-1, keepdims=True)
                p = jnp.exp(s)
                p = p / jnp.sum(p, axis=-1, keepdims=True)
                o = _dot(p.astype(f), v_ref[rows, cols], ((1,), (0,)))
                acc = acc + _dot(o.astype(f), wo_ref[cols, :], ((1,), (0,)))
            out_ref[b, :, :] = acc

    return pl.pallas_call(
        body,
        out_shape=jax.ShapeDtypeStruct((B, S, D), jnp.float32),
        in_specs=[pl.BlockSpec(memory_space=pltpu.VMEM)] * 8,
        out_specs=pl.BlockSpec(memory_space=pltpu.VMEM),
        scratch_shapes=[
            pltpu.VMEM((DC, BS), f),
            pltpu.VMEM((DC, D), f),
            pltpu.VMEM((DC, D), f),
            pltpu.VMEM((BS, D), f),
            pltpu.VMEM((BS, H * DR), f),
            pltpu.VMEM((BS, DR), f),
            pltpu.VMEM((BS, D), f),
            pltpu.VMEM((BS, D), f),
            pltpu.SemaphoreType.DMA((3, N_DEV - 1)),
            pltpu.SemaphoreType.DMA((3, N_DEV - 1)),
        ],
        compiler_params=_CompilerParams(
            collective_id=0, vmem_limit_bytes=64 * 1024 * 1024),
    )(xb, *(w.astype(f) for w in (Wdkv, Wuk, Wuv, Wq, Wqr, Wkr, Wo)))


# device time: 125781 ns/iter; 1.2485x vs baseline; 1.2485x over previous
import jax
import jax.numpy as jnp
from jax import lax
from jax.experimental import pallas as pl
from jax.experimental.pallas import tpu as pltpu

N_DEV = 4
B, S, D = 2, 512, 2048
H, DH, DR = 16, 128, 32
DC = 512
DCL = DC // N_DEV
BS = B * S
SCALE = (DH + DR) ** -0.5

_DevId = getattr(pl, "DeviceIdType", None) or pltpu.DeviceIdType
_sem_signal = getattr(pl, "semaphore_signal", None) or pltpu.semaphore_signal
_sem_wait = getattr(pl, "semaphore_wait", None) or pltpu.semaphore_wait
_CompilerParams = getattr(pltpu, "CompilerParams", None) or pltpu.TPUCompilerParams

CT, KU, VU = 0, 1, 2


def _dot(a, b, dims):
    return lax.dot_general(a, b, (dims, ((), ())),
                           preferred_element_type=jnp.float32)


def kernel(x, Wdkv, Wuk, Wuv, Wq, Wqr, Wkr, Wo):
    f = jnp.bfloat16
    xb = x.reshape(BS, D).astype(f)

    def body(x_ref, wdkv_ref, wuk_ref, wuv_ref, wq_ref, wqr_ref, wkr_ref,
             wo_ref, out_ref,
             ct_ref, ku_ref, vu_ref, q_ref, qr_ref, kr_ref,
             k_ref, v_ref, send_sems, recv_sems):
        my = lax.axis_index("i")
        others = [(my + d) % N_DEV for d in (1, 2, 3)]

        barrier = pltpu.get_barrier_semaphore()
        for nbr in others:
            _sem_signal(barrier, inc=1, device_id=(nbr,),
                        device_id_type=_DevId.MESH)
        _sem_wait(barrier, 3)

        xv = x_ref[...]
        ct = _dot(wdkv_ref[...], xv, ((0,), (1,))).astype(f)
        myrows = pl.ds(my * DCL, DCL)
        ct_ref[myrows] = ct
        ku_ref[myrows] = wuk_ref[...]
        vu_ref[myrows] = wuv_ref[...]

        def push(t, src, dst_buf, dest):
            r = pltpu.make_async_remote_copy(
                src_ref=src,
                dst_ref=dst_buf.at[myrows],
                send_sem=send_sems.at[t, dest],
                recv_sem=recv_sems.at[t, my],
                device_id=(dest,),
                device_id_type=_DevId.MESH,
            )
            r.start()
            return r

        def drain(t, buf, origin):
            orows = pl.ds(origin * DCL, DCL)
            pltpu.make_async_remote_copy(
                src_ref=buf.at[orows],
                dst_ref=buf.at[orows],
                send_sem=send_sems.at[t, origin],
                recv_sem=recv_sems.at[t, origin],
                device_id=(my,),
                device_id_type=_DevId.MESH,
            ).wait_recv()

        sends = []
        for dest in others:
            sends.append(push(CT, ct_ref.at[myrows], ct_ref, dest))
        for dest in others:
            sends.append(push(KU, wuk_ref, ku_ref, dest))

        q_ref[...] = _dot(xv, wq_ref[...], ((1,), (0,))).astype(f)
        qr_ref[...] = _dot(xv, wqr_ref[...], ((1,), (0,))).astype(f)
        kr_ref[...] = _dot(xv, wkr_ref[...], ((1,), (0,))).astype(f)

        for dest in others:
            sends.append(push(VU, wuv_ref, vu_ref, dest))

        for o in others:
            drain(CT, ct_ref, o)
        for o in others:
            drain(KU, ku_ref, o)
        k_ref[...] = _dot(ct_ref[...], ku_ref[...], ((0,), (0,))).astype(f)

        for o in others:
            drain(VU, vu_ref, o)
        v_ref[...] = _dot(ct_ref[...], vu_ref[...], ((0,), (0,))).astype(f)

        for b in range(B):
            rows = slice(b * S, (b + 1) * S)
            krb = kr_ref[rows, :]
            acc = jnp.zeros((S, D), jnp.float32)
            for hh in range(H):
                cols = slice(hh * DH, (hh + 1) * DH)
                rcols = slice(hh * DR, (hh + 1) * DR)
                s = _dot(q_ref[rows, cols], k_ref[rows, cols], ((1,), (1,)))
                s = s + _dot(qr_ref[rows, rcols], krb, ((1,), (1,)))
                e = jnp.exp(s * SCALE)
                denom = jnp.sum(e, axis=-1, keepdims=True)
                o = _dot(e.astype(f), v_ref[rows, cols], ((1,), (0,)))
                o = o * (1.0 / denom)
                acc = acc + _dot(o.astype(f), wo_ref[cols, :], ((1,), (0,)))
            out_ref[b, :, :] = acc

        for r in sends:
            r.wait_send()

    return pl.pallas_call(
        body,
        out_shape=jax.ShapeDtypeStruct((B, S, D), jnp.float32),
        in_specs=[pl.BlockSpec(memory_space=pltpu.VMEM)] * 8,
        out_specs=pl.BlockSpec(memory_space=pltpu.VMEM),
        scratch_shapes=[
            pltpu.VMEM((DC, BS), f),
            pltpu.VMEM((DC, D), f),
            pltpu.VMEM((DC, D), f),
            pltpu.VMEM((BS, D), f),
            pltpu.VMEM((BS, H * DR), f),
            pltpu.VMEM((BS, DR), f),
            pltpu.VMEM((BS, D), f),
            pltpu.VMEM((BS, D), f),
            pltpu.SemaphoreType.DMA((3, N_DEV)),
            pltpu.SemaphoreType.DMA((3, N_DEV)),
        ],
        compiler_params=_CompilerParams(
            collective_id=0, vmem_limit_bytes=64 * 1024 * 1024),
    )(xb, *(w.astype(f) for w in (Wdkv, Wuk, Wuv, Wq, Wqr, Wkr, Wo)))


# device time: 109424 ns/iter; 1.4352x vs baseline; 1.1495x over previous
import jax
import jax.numpy as jnp
from jax import lax
from jax.experimental import pallas as pl
from jax.experimental.pallas import tpu as pltpu

N_DEV = 4
B, S, D = 2, 512, 2048
H, DH, DR = 16, 128, 32
DC = 512
DCL = DC // N_DEV
BS = B * S
SCALE = (DH + DR) ** -0.5

_DevId = getattr(pl, "DeviceIdType", None) or pltpu.DeviceIdType
_sem_signal = getattr(pl, "semaphore_signal", None) or pltpu.semaphore_signal
_sem_wait = getattr(pl, "semaphore_wait", None) or pltpu.semaphore_wait
_CompilerParams = getattr(pltpu, "CompilerParams", None) or pltpu.TPUCompilerParams

CT, KU, VU = 0, 1, 2


def _dot(a, b, dims):
    return lax.dot_general(a, b, (dims, ((), ())),
                           preferred_element_type=jnp.float32)


def kernel(x, Wdkv, Wuk, Wuv, Wq, Wqr, Wkr, Wo):
    f = jnp.bfloat16
    xb = x.reshape(BS, D).astype(f)

    def body(x_ref, wdkv_ref, wuk_ref, wuv_ref, wq_ref, wqr_ref, wkr_ref,
             wo_ref, out_ref,
             ct_ref, ku_ref, vu_ref, q_ref, qr_ref, kr_ref,
             k_ref, v_ref, o_ref, e_ref, d_ref, send_sems, recv_sems):
        my = lax.axis_index("i")
        others = [(my + d) % N_DEV for d in (1, 2, 3)]

        barrier = pltpu.get_barrier_semaphore()
        for nbr in others:
            _sem_signal(barrier, inc=1, device_id=(nbr,),
                        device_id_type=_DevId.MESH)
        _sem_wait(barrier, 3)

        xv = x_ref[...]
        ct = _dot(wdkv_ref[...], xv, ((0,), (1,))).astype(f)
        myrows = pl.ds(my * DCL, DCL)
        ct_ref[myrows] = ct
        ku_ref[myrows] = wuk_ref[...]
        vu_ref[myrows] = wuv_ref[...]

        def push(t, src, dst_buf, dest):
            r = pltpu.make_async_remote_copy(
                src_ref=src,
                dst_ref=dst_buf.at[myrows],
                send_sem=send_sems.at[t, dest],
                recv_sem=recv_sems.at[t, my],
                device_id=(dest,),
                device_id_type=_DevId.MESH,
            )
            r.start()
            return r

        def drain(t, buf, origin):
            orows = pl.ds(origin * DCL, DCL)
            pltpu.make_async_remote_copy(
                src_ref=buf.at[orows],
                dst_ref=buf.at[orows],
                send_sem=send_sems.at[t, origin],
                recv_sem=recv_sems.at[t, origin],
                device_id=(my,),
                device_id_type=_DevId.MESH,
            ).wait_recv()

        sends = []
        for dest in others:
            sends.append(push(CT, ct_ref.at[myrows], ct_ref, dest))
        for dest in others:
            sends.append(push(KU, wuk_ref, ku_ref, dest))

        q_ref[...] = (_dot(xv, wq_ref[...], ((1,), (0,))) * SCALE).astype(f)
        qr_ref[...] = (_dot(xv, wqr_ref[...], ((1,), (0,))) * SCALE).astype(f)
        kr_ref[...] = _dot(xv, wkr_ref[...], ((1,), (0,))).astype(f)

        for dest in others:
            sends.append(push(VU, wuv_ref, vu_ref, dest))

        for o in others:
            drain(CT, ct_ref, o)
        for o in others:
            drain(KU, ku_ref, o)
        k_ref[...] = _dot(ct_ref[...], ku_ref[...], ((0,), (0,))).astype(f)

        def scores_exp(b, hh):
            rows = slice(b * S, (b + 1) * S)
            cols = slice(hh * DH, (hh + 1) * DH)
            rcols = slice(hh * DR, (hh + 1) * DR)
            s = _dot(q_ref[rows, cols], k_ref[rows, cols], ((1,), (1,)))
            s = s + _dot(qr_ref[rows, rcols], kr_ref[rows, :], ((1,), (1,)))
            return jnp.exp(s)

        def pv(e, recip, b, hh):
            rows = slice(b * S, (b + 1) * S)
            cols = slice(hh * DH, (hh + 1) * DH)
            o = _dot(e, v_ref[rows, cols], ((1,), (0,)))
            o_ref[:, cols] = (o * recip).astype(f)

        NA = 8
        for hh in range(NA):
            e = scores_exp(0, hh)
            e_ref[hh * S:(hh + 1) * S, :] = e.astype(f)
            d_ref[:, hh:hh + 1] = 1.0 / jnp.sum(e, axis=-1, keepdims=True)

        for o in others:
            drain(VU, vu_ref, o)
        v_ref[...] = _dot(ct_ref[...], vu_ref[...], ((0,), (0,))).astype(f)

        for b in range(B):
            for hh in range(H):
                if b == 0 and hh < NA:
                    e = e_ref[hh * S:(hh + 1) * S, :]
                    recip = d_ref[:, hh:hh + 1]
                else:
                    ef = scores_exp(b, hh)
                    recip = 1.0 / jnp.sum(ef, axis=-1, keepdims=True)
                    e = ef.astype(f)
                pv(e, recip, b, hh)
            out_ref[b, :, :] = _dot(o_ref[...], wo_ref[...], ((1,), (0,)))

        for r in sends:
            r.wait_send()

    return pl.pallas_call(
        body,
        out_shape=jax.ShapeDtypeStruct((B, S, D), jnp.float32),
        in_specs=[pl.BlockSpec(memory_space=pltpu.VMEM)] * 8,
        out_specs=pl.BlockSpec(memory_space=pltpu.VMEM),
        scratch_shapes=[
            pltpu.VMEM((DC, BS), f),
            pltpu.VMEM((DC, D), f),
            pltpu.VMEM((DC, D), f),
            pltpu.VMEM((BS, D), f),
            pltpu.VMEM((BS, H * DR), f),
            pltpu.VMEM((BS, DR), f),
            pltpu.VMEM((BS, D), f),
            pltpu.VMEM((BS, D), f),
            pltpu.VMEM((S, D), f),
            pltpu.VMEM((8 * S, S), f),
            pltpu.VMEM((S, 8), jnp.float32),
            pltpu.SemaphoreType.DMA((3, N_DEV)),
            pltpu.SemaphoreType.DMA((3, N_DEV)),
        ],
        compiler_params=_CompilerParams(
            collective_id=0, vmem_limit_bytes=64 * 1024 * 1024),
    )(xb, *(w.astype(f) for w in (Wdkv, Wuk, Wuv, Wq, Wqr, Wkr, Wo)))


# device time: 84829 ns/iter; 1.8513x vs baseline; 1.2899x over previous
import jax
import jax.numpy as jnp
from jax import lax
from jax.experimental import pallas as pl
from jax.experimental.pallas import tpu as pltpu

N_DEV = 4
B, S, D = 2, 512, 2048
H, DH, DR = 16, 128, 32
DC = 512
DCL = DC // N_DEV
BS = B * S
SCALE = (DH + DR) ** -0.5
NC, CW = 8, 256
NA = 8

_DevId = getattr(pl, "DeviceIdType", None) or pltpu.DeviceIdType
_sem_signal = getattr(pl, "semaphore_signal", None) or pltpu.semaphore_signal
_sem_wait = getattr(pl, "semaphore_wait", None) or pltpu.semaphore_wait
_CompilerParams = getattr(pltpu, "CompilerParams", None) or pltpu.TPUCompilerParams

CT, KU, VU = 0, 1, 2


def _dot(a, b, dims):
    return lax.dot_general(a, b, (dims, ((), ())),
                           preferred_element_type=jnp.float32)


def kernel(x, Wdkv, Wuk, Wuv, Wq, Wqr, Wkr, Wo):
    f = jnp.bfloat16

    def body(x_ref, wdkv_ref, wuk_ref, wuv_ref, wq_ref, wqr_ref, wkr_ref,
             wo_ref, out_ref,
             ct_ref, ku_ref, vu_ref, q_ref, qr_ref, kr_ref,
             k_ref, v_ref, o_ref, e_ref, d_ref, wstage_ref,
             send_sems, recv_sems, dma_sems):
        my = lax.axis_index("i")
        others = [(my + d) % N_DEV for d in (1, 2, 3)]

        barrier = pltpu.get_barrier_semaphore()
        for nbr in others:
            _sem_signal(barrier, inc=1, device_id=(nbr,),
                        device_id_type=_DevId.MESH)
        _sem_wait(barrier, 3)

        xv = x_ref[...].astype(f)
        ct = _dot(wdkv_ref[...].astype(f), xv, ((0,), (1,))).astype(f)
        myrows = pl.ds(my * DCL, DCL)
        ct_ref[myrows] = ct
        ku_ref[myrows] = wuk_ref[...].astype(f)
        vu_ref[myrows] = wuv_ref[...].astype(f)

        def push(t, buf, dest):
            r = pltpu.make_async_remote_copy(
                src_ref=buf.at[myrows],
                dst_ref=buf.at[myrows],
                send_sem=send_sems.at[t, dest],
                recv_sem=recv_sems.at[t, my],
                device_id=(dest,),
                device_id_type=_DevId.MESH,
            )
            r.start()
            return r

        def drain(t, buf, origin):
            orows = pl.ds(origin * DCL, DCL)
            pltpu.make_async_remote_copy(
                src_ref=buf.at[orows],
                dst_ref=buf.at[orows],
                send_sem=send_sems.at[t, origin],
                recv_sem=recv_sems.at[t, origin],
                device_id=(my,),
                device_id_type=_DevId.MESH,
            ).wait_recv()

        sends = []
        for dest in others:
            sends.append(push(CT, ct_ref, dest))
        for dest in others:
            sends.append(push(KU, ku_ref, dest))

        def stream_chunk(w_hbm, ci, slot):
            cp = pltpu.make_async_copy(
                w_hbm.at[:, pl.ds(ci * CW, CW)],
                wstage_ref.at[slot],
                dma_sems.at[slot],
            )
            cp.start()
            return cp

        cps = [stream_chunk(wq_ref, 0, 0), stream_chunk(wq_ref, 1, 1)]
        for ci in range(NC):
            slot = ci % 2
            cps[slot].wait()
            wb = wstage_ref[slot].astype(f)
            if ci + 2 < NC:
                cps[slot] = stream_chunk(wq_ref, ci + 2, slot)
            q_ref[:, ci * CW:(ci + 1) * CW] = (
                _dot(xv, wb, ((1,), (0,))) * SCALE).astype(f)
        qr_ref[...] = (_dot(xv, wqr_ref[...].astype(f), ((1,), (0,)))
                       * SCALE).astype(f)
        kr_ref[...] = _dot(xv, wkr_ref[...].astype(f), ((1,), (0,))).astype(f)

        for dest in others:
            sends.append(push(VU, vu_ref, dest))

        cps = [stream_chunk(wo_ref, 0, 0), stream_chunk(wo_ref, 1, 1)]

        for o in others:
            drain(CT, ct_ref, o)
        for o in others:
            drain(KU, ku_ref, o)
        k_ref[...] = _dot(ct_ref[...], ku_ref[...], ((0,), (0,))).astype(f)

        def scores_exp(b, hh):
            rows = slice(b * S, (b + 1) * S)
            cols = slice(hh * DH, (hh + 1) * DH)
            rcols = slice(hh * DR, (hh + 1) * DR)
            s = _dot(q_ref[rows, cols], k_ref[rows, cols], ((1,), (1,)))
            s = s + _dot(qr_ref[rows, rcols], kr_ref[rows, :], ((1,), (1,)))
            return jnp.exp(s)

        def pv(e, recip, b, hh):
            rows = slice(b * S, (b + 1) * S)
            cols = slice(hh * DH, (hh + 1) * DH)
            o = _dot(e, v_ref[rows, cols], ((1,), (0,)))
            o_ref[rows, cols] = (o * recip).astype(f)

        for hh in range(NA):
            e = scores_exp(0, hh)
            e_ref[hh * S:(hh + 1) * S, :] = e.astype(f)
            d_ref[:, hh:hh + 1] = 1.0 / jnp.sum(e, axis=-1, keepdims=True)

        for o in others:
            drain(VU, vu_ref, o)
        v_ref[...] = _dot(ct_ref[...], vu_ref[...], ((0,), (0,))).astype(f)

        for b in range(B):
            for hh in range(H):
                if b == 0 and hh < NA:
                    e = e_ref[hh * S:(hh + 1) * S, :]
                    recip = d_ref[:, hh:hh + 1]
                else:
                    ef = scores_exp(b, hh)
                    recip = 1.0 / jnp.sum(ef, axis=-1, keepdims=True)
                    e = ef.astype(f)
                pv(e, recip, b, hh)

        for ci in range(NC):
            slot = ci % 2
            cps[slot].wait()
            wb = wstage_ref[slot].astype(f)
            if ci + 2 < NC:
                cps[slot] = stream_chunk(wo_ref, ci + 2, slot)
            cols = slice(ci * CW, (ci + 1) * CW)
            out_ref[0, :, cols] = _dot(o_ref[0:S, :], wb, ((1,), (0,)))
            out_ref[1, :, cols] = _dot(o_ref[S:BS, :], wb, ((1,), (0,)))

        for r in sends:
            r.wait_send()

    vmem = pl.BlockSpec(memory_space=pltpu.VMEM)
    anym = pl.BlockSpec(memory_space=pl.ANY)
    return pl.pallas_call(
        body,
        out_shape=jax.ShapeDtypeStruct((B, S, D), jnp.float32),
        in_specs=[vmem, vmem, vmem, vmem, anym, vmem, vmem, anym],
        out_specs=vmem,
        scratch_shapes=[
            pltpu.VMEM((DC, BS), f),
            pltpu.VMEM((DC, D), f),
            pltpu.VMEM((DC, D), f),
            pltpu.VMEM((BS, D), f),
            pltpu.VMEM((BS, H * DR), f),
            pltpu.VMEM((BS, DR), f),
            pltpu.VMEM((BS, D), f),
            pltpu.VMEM((BS, D), f),
            pltpu.VMEM((BS, D), f),
            pltpu.VMEM((NA * S, S), f),
            pltpu.VMEM((S, NA), jnp.float32),
            pltpu.VMEM((2, D, CW), jnp.float32),
            pltpu.SemaphoreType.DMA((3, N_DEV)),
            pltpu.SemaphoreType.DMA((3, N_DEV)),
            pltpu.SemaphoreType.DMA((2,)),
        ],
        compiler_params=_CompilerParams(
            collective_id=0, vmem_limit_bytes=64 * 1024 * 1024),
    )(x.reshape(BS, D), Wdkv, Wuk, Wuv, Wq, Wqr, Wkr, Wo)


# device time: 81730 ns/iter; 1.9215x vs baseline; 1.0379x over previous
import jax
import jax.numpy as jnp
from jax import lax
from jax.experimental import pallas as pl
from jax.experimental.pallas import tpu as pltpu

N_DEV = 4
B, S, D = 2, 512, 2048
H, DH, DR = 16, 128, 32
DC = 512
DCL = DC // N_DEV
BS = B * S
SCALE = (DH + DR) ** -0.5
NC, CW = 8, 256
NA = 8

_DevId = getattr(pl, "DeviceIdType", None) or pltpu.DeviceIdType
_sem_signal = getattr(pl, "semaphore_signal", None) or pltpu.semaphore_signal
_sem_wait = getattr(pl, "semaphore_wait", None) or pltpu.semaphore_wait
_CompilerParams = getattr(pltpu, "CompilerParams", None) or pltpu.TPUCompilerParams

CT, KU, VU = 0, 1, 2


def _dot(a, b, dims):
    return lax.dot_general(a, b, (dims, ((), ())),
                           preferred_element_type=jnp.float32)


def kernel(x, Wdkv, Wuk, Wuv, Wq, Wqr, Wkr, Wo):
    f = jnp.bfloat16

    def body(x_ref, wdkv_ref, wuk_ref, wuv_ref, wq_ref, wqr_ref, wkr_ref,
             wo_ref, out_ref,
             ct_ref, ku_ref, vu_ref, q_ref, qr_ref, kr_ref,
             k_ref, v_ref, o_ref, e_ref, d_ref, wstage_ref,
             send_sems, recv_sems, dma_sems):
        my = lax.axis_index("i")
        others = [(my + d) % N_DEV for d in (1, 2, 3)]

        barrier = pltpu.get_barrier_semaphore()
        for nbr in others:
            _sem_signal(barrier, inc=1, device_id=(nbr,),
                        device_id_type=_DevId.MESH)
        _sem_wait(barrier, 3)

        xv = x_ref[...].astype(f)
        ct = _dot(wdkv_ref[...].astype(f), xv, ((0,), (1,))).astype(f)
        myrows = pl.ds(my * DCL, DCL)
        ct_ref[myrows] = ct
        ku_ref[myrows] = wuk_ref[...].astype(f)
        vu_ref[myrows] = wuv_ref[...].astype(f)

        def push(t, buf, dest):
            r = pltpu.make_async_remote_copy(
                src_ref=buf.at[myrows],
                dst_ref=buf.at[myrows],
                send_sem=send_sems.at[t, dest],
                recv_sem=recv_sems.at[t, my],
                device_id=(dest,),
                device_id_type=_DevId.MESH,
            )
            r.start()
            return r

        def drain(t, buf, origin):
            orows = pl.ds(origin * DCL, DCL)
            pltpu.make_async_remote_copy(
                src_ref=buf.at[orows],
                dst_ref=buf.at[orows],
                send_sem=send_sems.at[t, origin],
                recv_sem=recv_sems.at[t, origin],
                device_id=(my,),
                device_id_type=_DevId.MESH,
            ).wait_recv()

        sends = []
        for dest in others:
            sends.append(push(CT, ct_ref, dest))
        for dest in others:
            sends.append(push(KU, ku_ref, dest))

        def stream_chunk(w_hbm, ci, slot):
            cp = pltpu.make_async_copy(
                w_hbm.at[:, pl.ds(ci * CW, CW)],
                wstage_ref.at[slot],
                dma_sems.at[slot],
            )
            cp.start()
            return cp

        cps = [stream_chunk(wq_ref, 0, 0), stream_chunk(wq_ref, 1, 1)]
        for ci in range(NC):
            slot = ci % 2
            cps[slot].wait()
            wb = wstage_ref[slot].astype(f)
            if ci + 2 < NC:
                cps[slot] = stream_chunk(wq_ref, ci + 2, slot)
            q_ref[:, ci * CW:(ci + 1) * CW] = (
                _dot(xv, wb, ((1,), (0,))) * SCALE).astype(f)
        qr_ref[...] = (_dot(xv, wqr_ref[...].astype(f), ((1,), (0,)))
                       * SCALE).astype(f)
        kr_ref[...] = _dot(xv, wkr_ref[...].astype(f), ((1,), (0,))).astype(f)

        for dest in others:
            sends.append(push(VU, vu_ref, dest))

        cps = [stream_chunk(wo_ref, 0, 0), stream_chunk(wo_ref, 1, 1)]

        for o in others:
            drain(CT, ct_ref, o)
        for o in others:
            drain(KU, ku_ref, o)
        k_ref[...] = _dot(ct_ref[...], ku_ref[...], ((0,), (0,))).astype(f)

        def scores_exp(b, hh):
            rows = slice(b * S, (b + 1) * S)
            cols = slice(hh * DH, (hh + 1) * DH)
            rcols = slice(hh * DR, (hh + 1) * DR)
            s = _dot(q_ref[rows, cols], k_ref[rows, cols], ((1,), (1,)))
            s = s + _dot(qr_ref[rows, rcols], kr_ref[rows, :], ((1,), (1,)))
            return jnp.exp(s.astype(f))

        def _recip_rowsum(e):
            return 1.0 / jnp.sum(e, axis=-1, keepdims=True,
                                 dtype=jnp.float32)

        def pv(e, recip, b, hh):
            rows = slice(b * S, (b + 1) * S)
            cols = slice(hh * DH, (hh + 1) * DH)
            o = _dot(e, v_ref[rows, cols], ((1,), (0,)))
            o_ref[rows, cols] = (o * recip).astype(f)

        for hh in range(NA):
            e = scores_exp(0, hh)
            e_ref[hh * S:(hh + 1) * S, :] = e
            d_ref[:, hh:hh + 1] = _recip_rowsum(e)

        for o in others:
            drain(VU, vu_ref, o)
        v_ref[...] = _dot(ct_ref[...], vu_ref[...], ((0,), (0,))).astype(f)

        for b in range(B):
            for hh in range(H):
                if b == 0 and hh < NA:
                    e = e_ref[hh * S:(hh + 1) * S, :]
                    recip = d_ref[:, hh:hh + 1]
                else:
                    e = scores_exp(b, hh)
                    recip = _recip_rowsum(e)
                pv(e, recip, b, hh)

        for ci in range(NC):
            slot = ci % 2
            cps[slot].wait()
            wb = wstage_ref[slot].astype(f)
            if ci + 2 < NC:
                cps[slot] = stream_chunk(wo_ref, ci + 2, slot)
            cols = slice(ci * CW, (ci + 1) * CW)
            out_ref[0, :, cols] = _dot(o_ref[0:S, :], wb,
                                       ((1,), (0,))).astype(f)
            out_ref[1, :, cols] = _dot(o_ref[S:BS, :], wb,
                                       ((1,), (0,))).astype(f)

        for r in sends:
            r.wait_send()

    vmem = pl.BlockSpec(memory_space=pltpu.VMEM)
    anym = pl.BlockSpec(memory_space=pl.ANY)
    return pl.pallas_call(
        body,
        out_shape=jax.ShapeDtypeStruct((B, S, D), f),
        in_specs=[vmem, vmem, vmem, vmem, anym, vmem, vmem, anym],
        out_specs=vmem,
        scratch_shapes=[
            pltpu.VMEM((DC, BS), f),
            pltpu.VMEM((DC, D), f),
            pltpu.VMEM((DC, D), f),
            pltpu.VMEM((BS, D), f),
            pltpu.VMEM((BS, H * DR), f),
            pltpu.VMEM((BS, DR), f),
            pltpu.VMEM((BS, D), f),
            pltpu.VMEM((BS, D), f),
            pltpu.VMEM((BS, D), f),
            pltpu.VMEM((NA * S, S), f),
            pltpu.VMEM((S, NA), jnp.float32),
            pltpu.VMEM((2, D, CW), jnp.float32),
            pltpu.SemaphoreType.DMA((3, N_DEV)),
            pltpu.SemaphoreType.DMA((3, N_DEV)),
            pltpu.SemaphoreType.DMA((2,)),
        ],
        compiler_params=_CompilerParams(
            collective_id=0, vmem_limit_bytes=64 * 1024 * 1024),
    )(x.reshape(BS, D), Wdkv, Wuk, Wuv, Wq, Wqr, Wkr, Wo)


# device time: 81018 ns/iter; 1.9384x vs baseline; 1.0088x over previous
import jax
import jax.numpy as jnp
from jax import lax
from jax.experimental import pallas as pl
from jax.experimental.pallas import tpu as pltpu

N_DEV = 4
B, S, D = 2, 512, 2048
H, DH, DR = 16, 128, 32
DC = 512
DCL = DC // N_DEV
BS = B * S
SCALE = (DH + DR) ** -0.5
NC, CW = 8, 256
NS = 4
NA = 8

_DevId = getattr(pl, "DeviceIdType", None) or pltpu.DeviceIdType
_sem_signal = getattr(pl, "semaphore_signal", None) or pltpu.semaphore_signal
_sem_wait = getattr(pl, "semaphore_wait", None) or pltpu.semaphore_wait
_CompilerParams = getattr(pltpu, "CompilerParams", None) or pltpu.TPUCompilerParams

CT, KU, VU = 0, 1, 2


def _dot(a, b, dims):
    return lax.dot_general(a, b, (dims, ((), ())),
                           preferred_element_type=jnp.float32)


def kernel(x, Wdkv, Wuk, Wuv, Wq, Wqr, Wkr, Wo):
    f = jnp.bfloat16

    def body(x_ref, wdkv_ref, wuk_ref, wuv_ref, wq_ref, wqr_ref, wkr_ref,
             wo_ref, out_ref,
             ct_ref, ku_ref, vu_ref, q_ref, qr_ref, kr_ref,
             k_ref, v_ref, o_ref, e_ref, d_ref, wstage_ref,
             send_sems, recv_sems, dma_sems):
        my = lax.axis_index("i")
        others = [(my + d) % N_DEV for d in (1, 2, 3)]

        barrier = pltpu.get_barrier_semaphore()
        for nbr in others:
            _sem_signal(barrier, inc=1, device_id=(nbr,),
                        device_id_type=_DevId.MESH)
        _sem_wait(barrier, 3)

        xv = x_ref[...].reshape(BS, D).astype(f)
        ct = _dot(wdkv_ref[...].astype(f), xv, ((0,), (1,))).astype(f)
        myrows = pl.ds(my * DCL, DCL)
        ct_ref[myrows] = ct
        ku_ref[myrows] = wuk_ref[...].astype(f)
        vu_ref[myrows] = wuv_ref[...].astype(f)

        def push(t, buf, dest):
            r = pltpu.make_async_remote_copy(
                src_ref=buf.at[myrows],
                dst_ref=buf.at[myrows],
                send_sem=send_sems.at[t, dest],
                recv_sem=recv_sems.at[t, my],
                device_id=(dest,),
                device_id_type=_DevId.MESH,
            )
            r.start()
            return r

        def drain(t, buf, origin):
            orows = pl.ds(origin * DCL, DCL)
            pltpu.make_async_remote_copy(
                src_ref=buf.at[orows],
                dst_ref=buf.at[orows],
                send_sem=send_sems.at[t, origin],
                recv_sem=recv_sems.at[t, origin],
                device_id=(my,),
                device_id_type=_DevId.MESH,
            ).wait_recv()

        sends = []
        for dest in others:
            sends.append(push(CT, ct_ref, dest))
        for dest in others:
            sends.append(push(KU, ku_ref, dest))

        def stream_chunk(w_hbm, ci, slot):
            cp = pltpu.make_async_copy(
                w_hbm.at[:, pl.ds(ci * CW, CW)],
                wstage_ref.at[slot],
                dma_sems.at[slot],
            )
            cp.start()
            return cp

        cps = [stream_chunk(wq_ref, ci, ci) for ci in range(NS)]
        for ci in range(NC):
            slot = ci % NS
            cps[slot].wait()
            wb = wstage_ref[slot].astype(f)
            if ci + NS < NC:
                cps[slot] = stream_chunk(wq_ref, ci + NS, slot)
            q_ref[:, ci * CW:(ci + 1) * CW] = (
                _dot(xv, wb, ((1,), (0,))) * SCALE).astype(f)
        qr_ref[...] = (_dot(xv, wqr_ref[...].astype(f), ((1,), (0,)))
                       * SCALE).astype(f)
        kr_ref[...] = _dot(xv, wkr_ref[...].astype(f), ((1,), (0,))).astype(f)

        for dest in others:
            sends.append(push(VU, vu_ref, dest))

        cps = [stream_chunk(wo_ref, ci, ci) for ci in range(NS)]

        for o in others:
            drain(CT, ct_ref, o)
        for o in others:
            drain(KU, ku_ref, o)
        k_ref[...] = _dot(ct_ref[...], ku_ref[...], ((0,), (0,))).astype(f)

        def scores_exp(b, hh):
            rows = slice(b * S, (b + 1) * S)
            cols = slice(hh * DH, (hh + 1) * DH)
            rcols = slice(hh * DR, (hh + 1) * DR)
            s = _dot(q_ref[rows, cols], k_ref[rows, cols], ((1,), (1,)))
            s = s + _dot(qr_ref[rows, rcols], kr_ref[rows, :], ((1,), (1,)))
            return jnp.exp(s.astype(f))

        def _recip_rowsum(e):
            return 1.0 / jnp.sum(e, axis=-1, keepdims=True,
                                 dtype=jnp.float32)

        def pv(e, recip, b, hh):
            rows = slice(b * S, (b + 1) * S)
            cols = slice(hh * DH, (hh + 1) * DH)
            o = _dot(e, v_ref[rows, cols], ((1,), (0,)))
            o_ref[rows, cols] = (o * recip).astype(f)

        for hh in range(NA):
            e = scores_exp(0, hh)
            e_ref[hh * S:(hh + 1) * S, :] = e
            d_ref[:, hh:hh + 1] = _recip_rowsum(e)

        for o in others:
            drain(VU, vu_ref, o)
        v_ref[...] = _dot(ct_ref[...], vu_ref[...], ((0,), (0,))).astype(f)

        for b in range(B):
            for hh in range(H):
                if b == 0 and hh < NA:
                    e = e_ref[hh * S:(hh + 1) * S, :]
                    recip = d_ref[:, hh:hh + 1]
                else:
                    e = scores_exp(b, hh)
                    recip = _recip_rowsum(e)
                pv(e, recip, b, hh)

        for ci in range(NC):
            slot = ci % NS
            cps[slot].wait()
            wb = wstage_ref[slot].astype(f)
            if ci + NS < NC:
                cps[slot] = stream_chunk(wo_ref, ci + NS, slot)
            cols = slice(ci * CW, (ci + 1) * CW)
            out_ref[0, :, cols] = _dot(o_ref[0:S, :], wb,
                                       ((1,), (0,))).astype(f)
            out_ref[1, :, cols] = _dot(o_ref[S:BS, :], wb,
                                       ((1,), (0,))).astype(f)

        for r in sends:
            r.wait_send()

    vmem = pl.BlockSpec(memory_space=pltpu.VMEM)
    anym = pl.BlockSpec(memory_space=pl.ANY)
    return pl.pallas_call(
        body,
        out_shape=jax.ShapeDtypeStruct((B, S, D), f),
        in_specs=[vmem, vmem, vmem, vmem, anym, vmem, vmem, anym],
        out_specs=vmem,
        scratch_shapes=[
            pltpu.VMEM((DC, BS), f),
            pltpu.VMEM((DC, D), f),
            pltpu.VMEM((DC, D), f),
            pltpu.VMEM((BS, D), f),
            pltpu.VMEM((BS, H * DR), f),
            pltpu.VMEM((BS, DR), f),
            pltpu.VMEM((BS, D), f),
            pltpu.VMEM((BS, D), f),
            pltpu.VMEM((BS, D), f),
            pltpu.VMEM((NA * S, S), f),
            pltpu.VMEM((S, NA), jnp.float32),
            pltpu.VMEM((NS, D, CW), jnp.float32),
            pltpu.SemaphoreType.DMA((3, N_DEV)),
            pltpu.SemaphoreType.DMA((3, N_DEV)),
            pltpu.SemaphoreType.DMA((NS,)),
        ],
        compiler_params=_CompilerParams(
            collective_id=0, vmem_limit_bytes=64 * 1024 * 1024),
    )(x, Wdkv, Wuk, Wuv, Wq, Wqr, Wkr, Wo)
